# Initial kernel scaffold; baseline (speedup 1.0000x reference)
import jax
import jax.numpy as jnp
from jax.experimental import pallas as pl


def kernel(x, adj, inxs, Wq, Wk, Wv, Wo, ln1_g, ln1_b, W1, b1, W2, b2, ln2_g, ln2_b):
    raise NotImplementedError("write your pallas kernel here")


# speedup vs baseline: 2.8514x; 2.8514x over previous
import functools
import math

import jax
import jax.numpy as jnp
from jax import lax
from jax.experimental import pallas as pl
from jax.experimental.pallas import tpu as pltpu
from jax.experimental.pallas import tpu_sc as plsc

_N = 8192
_D = 128
_K = 32
_DFF = int(_D * 1.5)

_NC = 2
_NS = 16
_NW = _NC * _NS
_NODES_W = _N // _NW
_CH_NODES = 8
_CH_PAIRS = _CH_NODES * _K
_N_CH = _NODES_W // _CH_NODES
_IDXW = 128


def _kv_body(x_ref, wk_ref, wv_ref, k_ref, v_ref):
    xb = x_ref[...]
    k_ref[...] = jnp.dot(xb, wk_ref[...], preferred_element_type=jnp.float32)
    v_ref[...] = jnp.dot(xb, wv_ref[...], preferred_element_type=jnp.float32)


def _project_kv(x, Wk, Wv):
    bp = 1024
    return pl.pallas_call(
        _kv_body,
        grid=(_N // bp,),
        in_specs=[
            pl.BlockSpec((bp, _D), lambda i: (i, 0)),
            pl.BlockSpec((_D, _D), lambda i: (0, 0)),
            pl.BlockSpec((_D, _D), lambda i: (0, 0)),
        ],
        out_specs=[
            pl.BlockSpec((bp, _D), lambda i: (i, 0)),
            pl.BlockSpec((bp, _D), lambda i: (i, 0)),
        ],
        out_shape=[
            jax.ShapeDtypeStruct((_N, _D), jnp.float32),
            jax.ShapeDtypeStruct((_N, _D), jnp.float32),
        ],
    )(x, Wk, Wv)


_ROWS_W = _NODES_W * _K // _IDXW
_CH_ROWS = _CH_PAIRS // _IDXW


def _sc_gather_kernel(k_hbm, v_hbm, inxs_hbm, adj_hbm,
                      kn_out, vn_out, adjn_out,
                      idx_all, aidx_v, kn_b, vn_b, adj_b,
                      sem_k, sem_v, sem_a):
    wid = lax.axis_index("s") * _NC + lax.axis_index("c")
    node0 = wid * _NODES_W
    pltpu.sync_copy(inxs_hbm.at[pl.ds(wid * _ROWS_W, _ROWS_W)], idx_all)

    def chunk_body(g, carry):
        nbase = node0 + g * _CH_NODES
        pair0 = nbase * _K
        row0 = g * _CH_ROWS
        for j in range(_CH_ROWS):
            for r in range(_IDXW // 16):
                node_id = nbase + (j * _IDXW + r * 16) // _K
                aidx_v[j, pl.ds(r * 16, 16)] = (
                    idx_all[row0 + j, pl.ds(r * 16, 16)] + node_id * _N)
        cps = []
        for j in range(_CH_ROWS):
            cps.append(pltpu.async_copy(
                k_hbm.at[idx_all.at[row0 + j]],
                kn_b.at[pl.ds(j * _IDXW, _IDXW)], sem_k))
            cps.append(pltpu.async_copy(
                v_hbm.at[idx_all.at[row0 + j]],
                vn_b.at[pl.ds(j * _IDXW, _IDXW)], sem_v))
            cps.append(pltpu.async_copy(
                adj_hbm.at[aidx_v.at[j]],
                adj_b.at[pl.ds(j * _IDXW, _IDXW)], sem_a))
        for cp in cps:
            cp.wait()
        pltpu.sync_copy(kn_b, kn_out.at[pl.ds(pair0, _CH_PAIRS)])
        pltpu.sync_copy(vn_b, vn_out.at[pl.ds(pair0, _CH_PAIRS)])
        pltpu.sync_copy(adj_b, adjn_out.at[pl.ds(pair0, _CH_PAIRS)])
        return carry

    lax.fori_loop(0, _N_CH, chunk_body, 0)


def _sc_gather(k, v, inxs2d, adj_flat):
    mesh = plsc.VectorSubcoreMesh(core_axis_name="c", subcore_axis_name="s",
                                  num_cores=_NC, num_subcores=_NS)
    fn = functools.partial(
        pl.kernel,
        out_type=(
            jax.ShapeDtypeStruct((_N * _K, _D), jnp.float32),
            jax.ShapeDtypeStruct((_N * _K, _D), jnp.float32),
            jax.ShapeDtypeStruct((_N * _K,), jnp.float32),
        ),
        mesh=mesh,
        scratch_types=(
            pltpu.VMEM((_ROWS_W, _IDXW), jnp.int32),
            pltpu.VMEM((_CH_ROWS, _IDXW), jnp.int32),
            pltpu.VMEM((_CH_PAIRS, _D), jnp.float32),
            pltpu.VMEM((_CH_PAIRS, _D), jnp.float32),
            pltpu.VMEM((_CH_PAIRS,), jnp.float32),
            pltpu.SemaphoreType.DMA,
            pltpu.SemaphoreType.DMA,
            pltpu.SemaphoreType.DMA,
        ),
    )(_sc_gather_kernel)
    return fn(k, v, inxs2d, adj_flat)


_INV_SQRT_D = 1.0 / math.sqrt(_D)


def _attn_body(x_ref, kn_ref, vn_ref, adjn_ref, wq_ref, wo_ref,
               l1g_ref, l1b_ref, w1_ref, b1_ref, w2_ref, b2_ref,
               l2g_ref, l2b_ref, out_ref, *, blk):
    xb = x_ref[...]
    q = jnp.dot(xb, wq_ref[...], preferred_element_type=jnp.float32)
    kn = kn_ref[...].reshape(blk, _K, _D)
    vn = vn_ref[...].reshape(blk, _K, _D)
    scores = jnp.sum(q[:, None, :] * kn, axis=-1) * _INV_SQRT_D
    mask = jnp.where(adjn_ref[...] > 0, 0.0, -1e22).astype(jnp.float32)
    s = scores + mask
    m = jnp.max(s, axis=-1, keepdims=True)
    e = jnp.exp(s - m)
    attn = e / jnp.sum(e, axis=-1, keepdims=True)
    att = jnp.sum(attn[:, :, None] * vn, axis=1)
    att = jnp.dot(att, wo_ref[...], preferred_element_type=jnp.float32)
    h = xb + jnp.maximum(att, 0.0)
    mu = jnp.mean(h, axis=-1, keepdims=True)
    var = jnp.mean((h - mu) ** 2, axis=-1, keepdims=True)
    h = (h - mu) / jnp.sqrt(var + 1e-5) * l1g_ref[...] + l1b_ref[...]
    f = jnp.maximum(
        jnp.dot(h, w1_ref[...], preferred_element_type=jnp.float32)
        + b1_ref[...], 0.0)
    f = jnp.dot(f, w2_ref[...], preferred_element_type=jnp.float32) + b2_ref[...]
    h2 = h + f
    mu2 = jnp.mean(h2, axis=-1, keepdims=True)
    var2 = jnp.mean((h2 - mu2) ** 2, axis=-1, keepdims=True)
    out_ref[...] = ((h2 - mu2) / jnp.sqrt(var2 + 1e-5) * l2g_ref[...]
                    + l2b_ref[...])


def _attn_ffn(x, kn_flat, vn_flat, adjn, Wq, Wo, ln1_g, ln1_b,
              W1, b1, W2, b2, ln2_g, ln2_b, interpret=False):
    blk = 128
    const = lambda i: (0, 0)
    return pl.pallas_call(
        functools.partial(_attn_body, blk=blk),
        grid=(_N // blk,),
        in_specs=[
            pl.BlockSpec((blk, _D), lambda i: (i, 0)),
            pl.BlockSpec((blk * _K, _D), lambda i: (i, 0)),
            pl.BlockSpec((blk * _K, _D), lambda i: (i, 0)),
            pl.BlockSpec((blk, _K), lambda i: (i, 0)),
            pl.BlockSpec((_D, _D), const),
            pl.BlockSpec((_D, _D), const),
            pl.BlockSpec((1, _D), const),
            pl.BlockSpec((1, _D), const),
            pl.BlockSpec((_D, _DFF), const),
            pl.BlockSpec((1, _DFF), const),
            pl.BlockSpec((_DFF, _D), const),
            pl.BlockSpec((1, _D), const),
            pl.BlockSpec((1, _D), const),
            pl.BlockSpec((1, _D), const),
        ],
        out_specs=pl.BlockSpec((blk, _D), lambda i: (i, 0)),
        out_shape=jax.ShapeDtypeStruct((_N, _D), jnp.float32),
        interpret=interpret,
    )(x, kn_flat, vn_flat, adjn, Wq, Wo, ln1_g, ln1_b,
      W1, b1, W2, b2, ln2_g, ln2_b)


def kernel(x, adj, inxs, Wq, Wk, Wv, Wo, ln1_g, ln1_b, W1, b1, W2, b2,
           ln2_g, ln2_b):
    adj = jnp.squeeze(adj)
    k, v = _project_kv(x, Wk, Wv)
    inxs2d = inxs.astype(jnp.int32).reshape(_N * _K // _IDXW, _IDXW)
    adj_flat = adj.reshape(_N * _N)
    kn_flat, vn_flat, adjn_flat = _sc_gather(k, v, inxs2d, adj_flat)
    adjn = adjn_flat.reshape(_N, _K)
    return _attn_ffn(x, kn_flat, vn_flat, adjn,
                     Wq, Wo, ln1_g.reshape(1, _D), ln1_b.reshape(1, _D),
                     W1, b1.reshape(1, _DFF), W2, b2.reshape(1, _D),
                     ln2_g.reshape(1, _D), ln2_b.reshape(1, _D))


# speedup vs baseline: 2.8904x; 1.0137x over previous
import functools
import math

import jax
import jax.numpy as jnp
from jax import lax
from jax.experimental import pallas as pl
from jax.experimental.pallas import tpu as pltpu
from jax.experimental.pallas import tpu_sc as plsc

_N = 8192
_D = 128
_K = 32
_DFF = int(_D * 1.5)

_NC = 2
_NS = 16
_NW = _NC * _NS
_NODES_W = _N // _NW
_CH_NODES = 8
_CH_PAIRS = _CH_NODES * _K
_N_CH = _NODES_W // _CH_NODES
_IDXW = 128


def _kv_mask_body(x_ref, adj_ref, inxs_ref, wk_ref, wv_ref,
                  k_ref, v_ref, mask_ref, *, blk):
    xb = x_ref[...]
    k_ref[...] = jnp.dot(xb, wk_ref[...], preferred_element_type=jnp.float32)
    v_ref[...] = jnp.dot(xb, wv_ref[...], preferred_element_type=jnp.float32)
    ix = inxs_ref[...]
    lo = ix & (_D - 1)
    hi = ix >> 7
    acc = jnp.zeros((blk, _K), jnp.float32)
    for c in range(_N // _D):
        g = jnp.take_along_axis(adj_ref[:, c * _D:(c + 1) * _D], lo, axis=-1)
        acc = jnp.where(hi == c, g, acc)
    mask_ref[...] = jnp.where(acc > 0, 0.0, -1e22).astype(jnp.float32)


def _project_kv_mask(x, adj, inxs, Wk, Wv):
    blk = 128
    return pl.pallas_call(
        functools.partial(_kv_mask_body, blk=blk),
        grid=(_N // blk,),
        in_specs=[
            pl.BlockSpec((blk, _D), lambda i: (i, 0)),
            pl.BlockSpec((blk, _N), lambda i: (i, 0)),
            pl.BlockSpec((blk, _K), lambda i: (i, 0)),
            pl.BlockSpec((_D, _D), lambda i: (0, 0)),
            pl.BlockSpec((_D, _D), lambda i: (0, 0)),
        ],
        out_specs=[
            pl.BlockSpec((blk, _D), lambda i: (i, 0)),
            pl.BlockSpec((blk, _D), lambda i: (i, 0)),
            pl.BlockSpec((blk, _K), lambda i: (i, 0)),
        ],
        out_shape=[
            jax.ShapeDtypeStruct((_N, _D), jnp.float32),
            jax.ShapeDtypeStruct((_N, _D), jnp.float32),
            jax.ShapeDtypeStruct((_N, _K), jnp.float32),
        ],
    )(x, adj, inxs, Wk, Wv)


_ROWS_W = _NODES_W * _K // _IDXW
_CH_ROWS = _CH_PAIRS // _IDXW


def _sc_gather_kernel(k_hbm, v_hbm, inxs_hbm,
                      kn_out, vn_out,
                      idx_all, kn_b, vn_b,
                      sem_k, sem_v):
    wid = lax.axis_index("s") * _NC + lax.axis_index("c")
    node0 = wid * _NODES_W
    pltpu.sync_copy(inxs_hbm.at[pl.ds(wid * _ROWS_W, _ROWS_W)], idx_all)

    def chunk_body(g, carry):
        nbase = node0 + g * _CH_NODES
        pair0 = nbase * _K
        row0 = g * _CH_ROWS
        cps = []
        for j in range(_CH_ROWS):
            cps.append(pltpu.async_copy(
                k_hbm.at[idx_all.at[row0 + j]],
                kn_b.at[pl.ds(j * _IDXW, _IDXW)], sem_k))
            cps.append(pltpu.async_copy(
                v_hbm.at[idx_all.at[row0 + j]],
                vn_b.at[pl.ds(j * _IDXW, _IDXW)], sem_v))
        for cp in cps:
            cp.wait()
        pltpu.sync_copy(kn_b, kn_out.at[pl.ds(pair0, _CH_PAIRS)])
        pltpu.sync_copy(vn_b, vn_out.at[pl.ds(pair0, _CH_PAIRS)])
        return carry

    lax.fori_loop(0, _N_CH, chunk_body, 0)


def _sc_gather(k, v, inxs2d):
    mesh = plsc.VectorSubcoreMesh(core_axis_name="c", subcore_axis_name="s",
                                  num_cores=_NC, num_subcores=_NS)
    fn = functools.partial(
        pl.kernel,
        out_type=(
            jax.ShapeDtypeStruct((_N * _K, _D), jnp.float32),
            jax.ShapeDtypeStruct((_N * _K, _D), jnp.float32),
        ),
        mesh=mesh,
        scratch_types=(
            pltpu.VMEM((_ROWS_W, _IDXW), jnp.int32),
            pltpu.VMEM((_CH_PAIRS, _D), jnp.float32),
            pltpu.VMEM((_CH_PAIRS, _D), jnp.float32),
            pltpu.SemaphoreType.DMA,
            pltpu.SemaphoreType.DMA,
        ),
    )(_sc_gather_kernel)
    return fn(k, v, inxs2d)


_INV_SQRT_D = 1.0 / math.sqrt(_D)


def _attn_body(x_ref, kn_ref, vn_ref, mask_ref, wq_ref, wo_ref,
               l1g_ref, l1b_ref, w1_ref, b1_ref, w2_ref, b2_ref,
               l2g_ref, l2b_ref, out_ref, *, blk):
    xb = x_ref[...]
    q = jnp.dot(xb, wq_ref[...], preferred_element_type=jnp.float32)
    kn = kn_ref[...].reshape(blk, _K, _D)
    vn = vn_ref[...].reshape(blk, _K, _D)
    scores = jnp.sum(q[:, None, :] * kn, axis=-1) * _INV_SQRT_D
    s = scores + mask_ref[...]
    m = jnp.max(s, axis=-1, keepdims=True)
    e = jnp.exp(s - m)
    attn = e / jnp.sum(e, axis=-1, keepdims=True)
    att = jnp.sum(attn[:, :, None] * vn, axis=1)
    att = jnp.dot(att, wo_ref[...], preferred_element_type=jnp.float32)
    h = xb + jnp.maximum(att, 0.0)
    mu = jnp.mean(h, axis=-1, keepdims=True)
    var = jnp.mean((h - mu) ** 2, axis=-1, keepdims=True)
    h = (h - mu) / jnp.sqrt(var + 1e-5) * l1g_ref[...] + l1b_ref[...]
    f = jnp.maximum(
        jnp.dot(h, w1_ref[...], preferred_element_type=jnp.float32)
        + b1_ref[...], 0.0)
    f = jnp.dot(f, w2_ref[...], preferred_element_type=jnp.float32) + b2_ref[...]
    h2 = h + f
    mu2 = jnp.mean(h2, axis=-1, keepdims=True)
    var2 = jnp.mean((h2 - mu2) ** 2, axis=-1, keepdims=True)
    out_ref[...] = ((h2 - mu2) / jnp.sqrt(var2 + 1e-5) * l2g_ref[...]
                    + l2b_ref[...])


def _attn_ffn(x, kn_flat, vn_flat, mask, Wq, Wo, ln1_g, ln1_b,
              W1, b1, W2, b2, ln2_g, ln2_b, interpret=False):
    blk = 128
    const = lambda i: (0, 0)
    return pl.pallas_call(
        functools.partial(_attn_body, blk=blk),
        grid=(_N // blk,),
        in_specs=[
            pl.BlockSpec((blk, _D), lambda i: (i, 0)),
            pl.BlockSpec((blk * _K, _D), lambda i: (i, 0)),
            pl.BlockSpec((blk * _K, _D), lambda i: (i, 0)),
            pl.BlockSpec((blk, _K), lambda i: (i, 0)),
            pl.BlockSpec((_D, _D), const),
            pl.BlockSpec((_D, _D), const),
            pl.BlockSpec((1, _D), const),
            pl.BlockSpec((1, _D), const),
            pl.BlockSpec((_D, _DFF), const),
            pl.BlockSpec((1, _DFF), const),
            pl.BlockSpec((_DFF, _D), const),
            pl.BlockSpec((1, _D), const),
            pl.BlockSpec((1, _D), const),
            pl.BlockSpec((1, _D), const),
        ],
        out_specs=pl.BlockSpec((blk, _D), lambda i: (i, 0)),
        out_shape=jax.ShapeDtypeStruct((_N, _D), jnp.float32),
        interpret=interpret,
    )(x, kn_flat, vn_flat, mask, Wq, Wo, ln1_g, ln1_b,
      W1, b1, W2, b2, ln2_g, ln2_b)


def kernel(x, adj, inxs, Wq, Wk, Wv, Wo, ln1_g, ln1_b, W1, b1, W2, b2,
           ln2_g, ln2_b):
    adj = jnp.squeeze(adj)
    inxs = inxs.astype(jnp.int32)
    k, v, mask = _project_kv_mask(x, adj, inxs, Wk, Wv)
    inxs2d = inxs.reshape(_N * _K // _IDXW, _IDXW)
    kn_flat, vn_flat = _sc_gather(k, v, inxs2d)
    return _attn_ffn(x, kn_flat, vn_flat, mask,
                     Wq, Wo, ln1_g.reshape(1, _D), ln1_b.reshape(1, _D),
                     W1, b1.reshape(1, _DFF), W2, b2.reshape(1, _D),
                     ln2_g.reshape(1, _D), ln2_b.reshape(1, _D))


# speedup vs baseline: 3.9779x; 1.3762x over previous
import functools
import math

import jax
import jax.numpy as jnp
from jax import lax
from jax.experimental import pallas as pl
from jax.experimental.pallas import tpu as pltpu
from jax.experimental.pallas import tpu_sc as plsc

_N = 8192
_D = 128
_K = 32
_DFF = int(_D * 1.5)

_NC = 2
_NS = 16
_NW = _NC * _NS
_NODES_W = _N // _NW
_CH_NODES = 4
_CH_PAIRS = _CH_NODES * _K
_N_CH = _NODES_W // _CH_NODES
_IDXW = 128


def _kv_body(x_ref, wk_ref, wv_ref, k_ref, v_ref):
    xb = x_ref[...]
    k_ref[...] = jnp.dot(xb, wk_ref[...], preferred_element_type=jnp.float32)
    v_ref[...] = jnp.dot(xb, wv_ref[...], preferred_element_type=jnp.float32)


def _project_kv(x, Wk, Wv):
    bp = 1024
    return pl.pallas_call(
        _kv_body,
        grid=(_N // bp,),
        in_specs=[
            pl.BlockSpec((bp, _D), lambda i: (i, 0)),
            pl.BlockSpec((_D, _D), lambda i: (0, 0)),
            pl.BlockSpec((_D, _D), lambda i: (0, 0)),
        ],
        out_specs=[
            pl.BlockSpec((bp, _D), lambda i: (i, 0)),
            pl.BlockSpec((bp, _D), lambda i: (i, 0)),
        ],
        out_shape=[
            jax.ShapeDtypeStruct((_N, _D), jnp.float32),
            jax.ShapeDtypeStruct((_N, _D), jnp.float32),
        ],
    )(x, Wk, Wv)


_ROWS_W = _NODES_W * _K // _IDXW


def _sc_gather_kernel(k_hbm, v_hbm, inxs_hbm,
                      kn_out, vn_out,
                      idx_all, kn_b0, vn_b0, kn_b1, vn_b1,
                      sem_k0, sem_v0, sem_k1, sem_v1):
    wid = lax.axis_index("s") * _NC + lax.axis_index("c")
    node0 = wid * _NODES_W
    pltpu.sync_copy(inxs_hbm.at[pl.ds(wid * _ROWS_W, _ROWS_W)], idx_all)

    kn_bufs = (kn_b0, kn_b1)
    vn_bufs = (vn_b0, vn_b1)
    sems = ((sem_k0, sem_v0), (sem_k1, sem_v1))

    def start(g, slot):
        ck = pltpu.make_async_copy(
            k_hbm.at[idx_all.at[g]], kn_bufs[slot], sems[slot][0])
        cv = pltpu.make_async_copy(
            v_hbm.at[idx_all.at[g]], vn_bufs[slot], sems[slot][1])
        ck.start()
        cv.start()
        return ck, cv

    prime = start(0, 0)

    def outer(gg, carry):
        for s in range(2):
            g = gg * 2 + s
            ck = pltpu.make_async_copy(
                k_hbm.at[idx_all.at[g]], kn_bufs[s], sems[s][0])
            cv = pltpu.make_async_copy(
                v_hbm.at[idx_all.at[g]], vn_bufs[s], sems[s][1])
            ck.wait()
            cv.wait()

            @pl.when(g + 1 < _N_CH)
            def _():
                nxt = (s + 1) % 2
                pltpu.make_async_copy(
                    k_hbm.at[idx_all.at[g + 1]], kn_bufs[nxt],
                    sems[nxt][0]).start()
                pltpu.make_async_copy(
                    v_hbm.at[idx_all.at[g + 1]], vn_bufs[nxt],
                    sems[nxt][1]).start()

            pair0 = (node0 + g * _CH_NODES) * _K
            pltpu.sync_copy(kn_bufs[s], kn_out.at[pl.ds(pair0, _CH_PAIRS)])
            pltpu.sync_copy(vn_bufs[s], vn_out.at[pl.ds(pair0, _CH_PAIRS)])
        return carry

    lax.fori_loop(0, _N_CH // 2, outer, 0)


def _sc_gather(k, v, inxs2d):
    mesh = plsc.VectorSubcoreMesh(core_axis_name="c", subcore_axis_name="s",
                                  num_cores=_NC, num_subcores=_NS)
    fn = functools.partial(
        pl.kernel,
        out_type=(
            jax.ShapeDtypeStruct((_N * _K, _D), jnp.float32),
            jax.ShapeDtypeStruct((_N * _K, _D), jnp.float32),
        ),
        mesh=mesh,
        scratch_types=(
            pltpu.VMEM((_ROWS_W, _IDXW), jnp.int32),
            pltpu.VMEM((_CH_PAIRS, _D), jnp.float32),
            pltpu.VMEM((_CH_PAIRS, _D), jnp.float32),
            pltpu.VMEM((_CH_PAIRS, _D), jnp.float32),
            pltpu.VMEM((_CH_PAIRS, _D), jnp.float32),
            pltpu.SemaphoreType.DMA,
            pltpu.SemaphoreType.DMA,
            pltpu.SemaphoreType.DMA,
            pltpu.SemaphoreType.DMA,
        ),
    )(_sc_gather_kernel)
    return fn(k, v, inxs2d)


def _mask_body(adj_ref, inxs_ref, mask_ref, *, blk):
    ix = inxs_ref[...]
    lo = ix & (_D - 1)
    hi = ix >> 7
    acc = jnp.zeros((blk, _K), jnp.float32)
    for p in range(4):
        packed = jnp.zeros((blk, _D), jnp.float32)
        for m in range(16):
            c = p * 16 + m
            sg = adj_ref[:, c * _D:(c + 1) * _D] > 0
            packed = packed + jnp.where(sg, float(1 << m), 0.0)
        g = jnp.take_along_axis(packed, lo, axis=-1)
        acc = jnp.where((hi >> 4) == p, g, acc)
    bits = (acc.astype(jnp.int32) >> (hi & 15)) & 1
    mask_ref[...] = jnp.where(bits == 1, 0.0, -1e22).astype(jnp.float32)


def _mask_extract(adj, inxs, interpret=False):
    blk = 128
    return pl.pallas_call(
        functools.partial(_mask_body, blk=blk),
        grid=(_N // blk,),
        in_specs=[
            pl.BlockSpec((blk, _N), lambda i: (i, 0)),
            pl.BlockSpec((blk, _K), lambda i: (i, 0)),
        ],
        out_specs=pl.BlockSpec((blk, _K), lambda i: (i, 0)),
        out_shape=jax.ShapeDtypeStruct((_N, _K), jnp.float32),
        interpret=interpret,
    )(adj, inxs)


_INV_SQRT_D = 1.0 / math.sqrt(_D)


def _attn_body(x_ref, kn_ref, vn_ref, mask_ref, rep_ref, rept_ref, oh_ref,
               wq_ref, wo_ref, l1g_ref, l1b_ref, w1_ref, b1_ref,
               w2_ref, b2_ref, l2g_ref, l2b_ref, out_ref, *, blk):
    f32 = jnp.float32
    dot = functools.partial(jnp.dot, preferred_element_type=f32)
    xb = x_ref[...]
    q = dot(xb, wq_ref[...])
    rep = rep_ref[...]
    rept = rept_ref[...]
    oh = oh_ref[...]
    ones_d_k = jnp.ones((_D, _K), f32)
    ones_k_d = jnp.ones((_K, _D), f32)
    q_rep = dot(rep, q)
    e = q_rep * kn_ref[...]
    e1 = dot(e, ones_d_k)
    scores = dot(rept, e1 * oh) * _INV_SQRT_D
    s = scores + mask_ref[...]
    m = jnp.max(s, axis=-1, keepdims=True)
    ex = jnp.exp(s - m)
    attn = ex / jnp.sum(ex, axis=-1, keepdims=True)
    a1 = dot(rep, attn)
    attn_rep = dot(a1 * oh, ones_k_d)
    w = attn_rep * vn_ref[...]
    att = jnp.sum(w.reshape(blk, _K, _D), axis=1)
    att = dot(att, wo_ref[...])
    h = xb + jnp.maximum(att, 0.0)
    mu = jnp.mean(h, axis=-1, keepdims=True)
    var = jnp.mean((h - mu) ** 2, axis=-1, keepdims=True)
    h = (h - mu) / jnp.sqrt(var + 1e-5) * l1g_ref[...] + l1b_ref[...]
    f = jnp.maximum(dot(h, w1_ref[...]) + b1_ref[...], 0.0)
    f = dot(f, w2_ref[...]) + b2_ref[...]
    h2 = h + f
    mu2 = jnp.mean(h2, axis=-1, keepdims=True)
    var2 = jnp.mean((h2 - mu2) ** 2, axis=-1, keepdims=True)
    out_ref[...] = ((h2 - mu2) / jnp.sqrt(var2 + 1e-5) * l2g_ref[...]
                    + l2b_ref[...])


def _attn_ffn(x, kn_flat, vn_flat, mask, Wq, Wo, ln1_g, ln1_b,
              W1, b1, W2, b2, ln2_g, ln2_b, interpret=False):
    blk = 128
    r = jnp.arange(blk * _K, dtype=jnp.int32)
    rep = (r[:, None] // _K == jnp.arange(blk)[None, :]).astype(jnp.float32)
    rept = rep.T
    oh = (r[:, None] % _K == jnp.arange(_K)[None, :]).astype(jnp.float32)
    const = lambda i: (0, 0)
    return pl.pallas_call(
        functools.partial(_attn_body, blk=blk),
        grid=(_N // blk,),
        in_specs=[
            pl.BlockSpec((blk, _D), lambda i: (i, 0)),
            pl.BlockSpec((blk * _K, _D), lambda i: (i, 0)),
            pl.BlockSpec((blk * _K, _D), lambda i: (i, 0)),
            pl.BlockSpec((blk, _K), lambda i: (i, 0)),
            pl.BlockSpec((blk * _K, blk), const),
            pl.BlockSpec((blk, blk * _K), const),
            pl.BlockSpec((blk * _K, _K), const),
            pl.BlockSpec((_D, _D), const),
            pl.BlockSpec((_D, _D), const),
            pl.BlockSpec((1, _D), const),
            pl.BlockSpec((1, _D), const),
            pl.BlockSpec((_D, _DFF), const),
            pl.BlockSpec((1, _DFF), const),
            pl.BlockSpec((_DFF, _D), const),
            pl.BlockSpec((1, _D), const),
            pl.BlockSpec((1, _D), const),
            pl.BlockSpec((1, _D), const),
        ],
        out_specs=pl.BlockSpec((blk, _D), lambda i: (i, 0)),
        out_shape=jax.ShapeDtypeStruct((_N, _D), jnp.float32),
        interpret=interpret,
    )(x, kn_flat, vn_flat, mask, rep, rept, oh, Wq, Wo, ln1_g, ln1_b,
      W1, b1, W2, b2, ln2_g, ln2_b)


def kernel(x, adj, inxs, Wq, Wk, Wv, Wo, ln1_g, ln1_b, W1, b1, W2, b2,
           ln2_g, ln2_b):
    adj = jnp.squeeze(adj)
    inxs = inxs.astype(jnp.int32)
    k, v = _project_kv(x, Wk, Wv)
    inxs2d = inxs.reshape(_N * _K // _IDXW, _IDXW)
    kn_flat, vn_flat = _sc_gather(k, v, inxs2d)
    mask = _mask_extract(adj, inxs)
    return _attn_ffn(x, kn_flat, vn_flat, mask,
                     Wq, Wo, ln1_g.reshape(1, _D), ln1_b.reshape(1, _D),
                     W1, b1.reshape(1, _DFF), W2, b2.reshape(1, _D),
                     ln2_g.reshape(1, _D), ln2_b.reshape(1, _D))


# speedup vs baseline: 4.7945x; 1.2053x over previous
import functools
import math

import jax
import jax.numpy as jnp
from jax import lax
from jax.experimental import pallas as pl
from jax.experimental.pallas import tpu as pltpu
from jax.experimental.pallas import tpu_sc as plsc

_N = 8192
_D = 128
_K = 32
_DFF = int(_D * 1.5)

_NC = 2
_NS = 16
_NW = _NC * _NS
_NODES_W = _N // _NW
_CH_NODES = 8
_CH_PAIRS = _CH_NODES * _K
_N_CH = _NODES_W // _CH_NODES
_IDXW = 128


def _kv_body(x_ref, wk_ref, wv_ref, kv_ref):
    xb = x_ref[...]
    k = jnp.dot(xb, wk_ref[...], preferred_element_type=jnp.float32)
    v = jnp.dot(xb, wv_ref[...], preferred_element_type=jnp.float32)
    ki = lax.bitcast_convert_type(k.astype(jnp.bfloat16),
                                  jnp.int16).astype(jnp.int32)
    vi = lax.bitcast_convert_type(v.astype(jnp.bfloat16),
                                  jnp.int16).astype(jnp.int32)
    kv_ref[...] = (ki & 0xFFFF) | (vi << 16)


def _project_kv(x, Wk, Wv):
    bp = 1024
    return pl.pallas_call(
        _kv_body,
        grid=(_N // bp,),
        in_specs=[
            pl.BlockSpec((bp, _D), lambda i: (i, 0)),
            pl.BlockSpec((_D, _D), lambda i: (0, 0)),
            pl.BlockSpec((_D, _D), lambda i: (0, 0)),
        ],
        out_specs=pl.BlockSpec((bp, _D), lambda i: (i, 0)),
        out_shape=jax.ShapeDtypeStruct((_N, _D), jnp.int32),
    )(x, Wk, Wv)


_ROWS_W = _NODES_W * _K // _IDXW
_CH_ROWS = _CH_PAIRS // _IDXW


def _sc_gather_kernel(kv_hbm, inxs_hbm, kvn_out,
                      idx_all, b0, b1, sem0, sem1):
    wid = lax.axis_index("s") * _NC + lax.axis_index("c")
    node0 = wid * _NODES_W
    pltpu.sync_copy(inxs_hbm.at[pl.ds(wid * _ROWS_W, _ROWS_W)], idx_all)

    bufs = (b0, b1)
    sems = (sem0, sem1)

    def mk(g, slot):
        return [pltpu.make_async_copy(
            kv_hbm.at[idx_all.at[g * _CH_ROWS + j]],
            bufs[slot].at[pl.ds(j * _IDXW, _IDXW)],
            sems[slot]) for j in range(_CH_ROWS)]

    for cp in mk(0, 0):
        cp.start()

    def outer(gg, carry):
        for s in range(2):
            g = gg * 2 + s
            for cp in mk(g, s):
                cp.wait()

            @pl.when(g + 1 < _N_CH)
            def _():
                for cp in mk(g + 1, (s + 1) % 2):
                    cp.start()

            pair0 = (node0 + g * _CH_NODES) * _K
            pltpu.sync_copy(bufs[s], kvn_out.at[pl.ds(pair0, _CH_PAIRS)])
        return carry

    lax.fori_loop(0, _N_CH // 2, outer, 0)


def _sc_gather(kv, inxs2d):
    mesh = plsc.VectorSubcoreMesh(core_axis_name="c", subcore_axis_name="s",
                                  num_cores=_NC, num_subcores=_NS)
    fn = functools.partial(
        pl.kernel,
        out_type=jax.ShapeDtypeStruct((_N * _K, _D), jnp.int32),
        mesh=mesh,
        scratch_types=(
            pltpu.VMEM((_ROWS_W, _IDXW), jnp.int32),
            pltpu.VMEM((_CH_PAIRS, _D), jnp.int32),
            pltpu.VMEM((_CH_PAIRS, _D), jnp.int32),
            pltpu.SemaphoreType.DMA,
            pltpu.SemaphoreType.DMA,
        ),
    )(_sc_gather_kernel)
    return fn(kv, inxs2d)


def _mask_body(adj_ref, inxs_ref, mask_ref, *, blk):
    ix = inxs_ref[...]
    lo = ix & (_D - 1)
    hi = ix >> 7
    acc = jnp.zeros((blk, _K), jnp.float32)
    for p in range(4):
        packed = jnp.zeros((blk, _D), jnp.float32)
        for m in range(16):
            c = p * 16 + m
            sg = adj_ref[:, c * _D:(c + 1) * _D] > 0
            packed = packed + jnp.where(sg, float(1 << m), 0.0)
        g = jnp.take_along_axis(packed, lo, axis=-1)
        acc = jnp.where((hi >> 4) == p, g, acc)
    bits = (acc.astype(jnp.int32) >> (hi & 15)) & 1
    mask_ref[...] = jnp.where(bits == 1, 0.0, -1e22).astype(jnp.float32)


def _mask_extract(adj, inxs, interpret=False):
    blk = 128
    return pl.pallas_call(
        functools.partial(_mask_body, blk=blk),
        grid=(_N // blk,),
        in_specs=[
            pl.BlockSpec((blk, _N), lambda i: (i, 0)),
            pl.BlockSpec((blk, _K), lambda i: (i, 0)),
        ],
        out_specs=pl.BlockSpec((blk, _K), lambda i: (i, 0)),
        out_shape=jax.ShapeDtypeStruct((_N, _K), jnp.float32),
        interpret=interpret,
    )(adj, inxs)


_INV_SQRT_D = 1.0 / math.sqrt(_D)


def _attn_body(x_ref, kvn_ref, mask_ref, rep_ref, rept_ref, oh_ref,
               wq_ref, wo_ref, l1g_ref, l1b_ref, w1_ref, b1_ref,
               w2_ref, b2_ref, l2g_ref, l2b_ref, out_ref, *, blk):
    f32 = jnp.float32
    dot = functools.partial(jnp.dot, preferred_element_type=f32)
    xb = x_ref[...]
    q = dot(xb, wq_ref[...])
    wkv = kvn_ref[...]
    kn = lax.bitcast_convert_type(wkv << 16, f32)
    vn = lax.bitcast_convert_type(wkv & jnp.int32(-65536), f32)
    rep = rep_ref[...]
    rept = rept_ref[...]
    oh = oh_ref[...]
    ones_d_k = jnp.ones((_D, _K), f32)
    ones_k_d = jnp.ones((_K, _D), f32)
    q_rep = dot(rep, q)
    e = q_rep * kn
    e1 = dot(e, ones_d_k)
    scores = dot(rept, e1 * oh) * _INV_SQRT_D
    s = scores + mask_ref[...]
    m = jnp.max(s, axis=-1, keepdims=True)
    ex = jnp.exp(s - m)
    attn = ex / jnp.sum(ex, axis=-1, keepdims=True)
    a1 = dot(rep, attn)
    attn_rep = dot(a1 * oh, ones_k_d)
    w = attn_rep * vn
    att = jnp.sum(w.reshape(blk, _K, _D), axis=1)
    att = dot(att, wo_ref[...])
    h = xb + jnp.maximum(att, 0.0)
    mu = jnp.mean(h, axis=-1, keepdims=True)
    var = jnp.mean((h - mu) ** 2, axis=-1, keepdims=True)
    h = (h - mu) / jnp.sqrt(var + 1e-5) * l1g_ref[...] + l1b_ref[...]
    f = jnp.maximum(dot(h, w1_ref[...]) + b1_ref[...], 0.0)
    f = dot(f, w2_ref[...]) + b2_ref[...]
    h2 = h + f
    mu2 = jnp.mean(h2, axis=-1, keepdims=True)
    var2 = jnp.mean((h2 - mu2) ** 2, axis=-1, keepdims=True)
    out_ref[...] = ((h2 - mu2) / jnp.sqrt(var2 + 1e-5) * l2g_ref[...]
                    + l2b_ref[...])


def _attn_ffn(x, kvn_flat, mask, Wq, Wo, ln1_g, ln1_b,
              W1, b1, W2, b2, ln2_g, ln2_b, interpret=False):
    blk = 128
    r = jnp.arange(blk * _K, dtype=jnp.int32)
    rep = (r[:, None] // _K == jnp.arange(blk)[None, :]).astype(jnp.float32)
    rept = rep.T
    oh = (r[:, None] % _K == jnp.arange(_K)[None, :]).astype(jnp.float32)
    const = lambda i: (0, 0)
    return pl.pallas_call(
        functools.partial(_attn_body, blk=blk),
        grid=(_N // blk,),
        in_specs=[
            pl.BlockSpec((blk, _D), lambda i: (i, 0)),
            pl.BlockSpec((blk * _K, _D), lambda i: (i, 0)),
            pl.BlockSpec((blk, _K), lambda i: (i, 0)),
            pl.BlockSpec((blk * _K, blk), const),
            pl.BlockSpec((blk, blk * _K), const),
            pl.BlockSpec((blk * _K, _K), const),
            pl.BlockSpec((_D, _D), const),
            pl.BlockSpec((_D, _D), const),
            pl.BlockSpec((1, _D), const),
            pl.BlockSpec((1, _D), const),
            pl.BlockSpec((_D, _DFF), const),
            pl.BlockSpec((1, _DFF), const),
            pl.BlockSpec((_DFF, _D), const),
            pl.BlockSpec((1, _D), const),
            pl.BlockSpec((1, _D), const),
            pl.BlockSpec((1, _D), const),
        ],
        out_specs=pl.BlockSpec((blk, _D), lambda i: (i, 0)),
        out_shape=jax.ShapeDtypeStruct((_N, _D), jnp.float32),
        interpret=interpret,
    )(x, kvn_flat, mask, rep, rept, oh, Wq, Wo, ln1_g, ln1_b,
      W1, b1, W2, b2, ln2_g, ln2_b)


def kernel(x, adj, inxs, Wq, Wk, Wv, Wo, ln1_g, ln1_b, W1, b1, W2, b2,
           ln2_g, ln2_b):
    adj = jnp.squeeze(adj)
    inxs = inxs.astype(jnp.int32)
    kv = _project_kv(x, Wk, Wv)
    inxs2d = inxs.reshape(_N * _K // _IDXW, _IDXW)
    kvn_flat = _sc_gather(kv, inxs2d)
    mask = _mask_extract(adj, inxs)
    return _attn_ffn(x, kvn_flat, mask,
                     Wq, Wo, ln1_g.reshape(1, _D), ln1_b.reshape(1, _D),
                     W1, b1.reshape(1, _DFF), W2, b2.reshape(1, _D),
                     ln2_g.reshape(1, _D), ln2_b.reshape(1, _D))


# speedup vs baseline: 4.8013x; 1.0014x over previous
import functools
import math

import jax
import jax.numpy as jnp
from jax import lax
from jax.experimental import pallas as pl
from jax.experimental.pallas import tpu as pltpu
from jax.experimental.pallas import tpu_sc as plsc

_N = 8192
_D = 128
_K = 32
_DFF = int(_D * 1.5)

_NC = 2
_NS = 16
_NW = _NC * _NS
_NODES_W = _N // _NW
_CH_NODES = 8
_CH_PAIRS = _CH_NODES * _K
_N_CH = _NODES_W // _CH_NODES
_IDXW = 128


def _kv_body(x_ref, wk_ref, wv_ref, kv_ref):
    xb = x_ref[...]
    k = jnp.dot(xb, wk_ref[...], preferred_element_type=jnp.float32)
    v = jnp.dot(xb, wv_ref[...], preferred_element_type=jnp.float32)
    ki = lax.bitcast_convert_type(k.astype(jnp.bfloat16),
                                  jnp.int16).astype(jnp.int32)
    vi = lax.bitcast_convert_type(v.astype(jnp.bfloat16),
                                  jnp.int16).astype(jnp.int32)
    kv_ref[...] = (ki & 0xFFFF) | (vi << 16)


def _project_kv(x, Wk, Wv):
    bp = 1024
    return pl.pallas_call(
        _kv_body,
        grid=(_N // bp,),
        in_specs=[
            pl.BlockSpec((bp, _D), lambda i: (i, 0)),
            pl.BlockSpec((_D, _D), lambda i: (0, 0)),
            pl.BlockSpec((_D, _D), lambda i: (0, 0)),
        ],
        out_specs=pl.BlockSpec((bp, _D), lambda i: (i, 0)),
        out_shape=jax.ShapeDtypeStruct((_N, _D), jnp.int32),
    )(x, Wk, Wv)


_ROWS_W = _NODES_W * _K // _IDXW
_CH_ROWS = _CH_PAIRS // _IDXW


def _sc_gather_kernel(kv_hbm, inxs_hbm, kvn_out,
                      idx_all, b0, b1, sem0, sem1):
    wid = lax.axis_index("s") * _NC + lax.axis_index("c")
    node0 = wid * _NODES_W
    pltpu.sync_copy(inxs_hbm.at[pl.ds(wid * _ROWS_W, _ROWS_W)], idx_all)

    bufs = (b0, b1)
    sems = (sem0, sem1)

    def mk(g, slot):
        return [pltpu.make_async_copy(
            kv_hbm.at[idx_all.at[g * _CH_ROWS + j]],
            bufs[slot].at[pl.ds(j * _IDXW, _IDXW)],
            sems[slot]) for j in range(_CH_ROWS)]

    for cp in mk(0, 0):
        cp.start()

    def outer(gg, carry):
        for s in range(2):
            g = gg * 2 + s
            for cp in mk(g, s):
                cp.wait()

            @pl.when(g + 1 < _N_CH)
            def _():
                for cp in mk(g + 1, (s + 1) % 2):
                    cp.start()

            pair0 = (node0 + g * _CH_NODES) * _K
            pltpu.sync_copy(bufs[s], kvn_out.at[pl.ds(pair0, _CH_PAIRS)])
        return carry

    lax.fori_loop(0, _N_CH // 2, outer, 0)


def _sc_gather(kv, inxs2d):
    mesh = plsc.VectorSubcoreMesh(core_axis_name="c", subcore_axis_name="s",
                                  num_cores=_NC, num_subcores=_NS)
    fn = functools.partial(
        pl.kernel,
        out_type=jax.ShapeDtypeStruct((_N * _K, _D), jnp.int32),
        mesh=mesh,
        scratch_types=(
            pltpu.VMEM((_ROWS_W, _IDXW), jnp.int32),
            pltpu.VMEM((_CH_PAIRS, _D), jnp.int32),
            pltpu.VMEM((_CH_PAIRS, _D), jnp.int32),
            pltpu.SemaphoreType.DMA,
            pltpu.SemaphoreType.DMA,
        ),
    )(_sc_gather_kernel)
    return fn(kv, inxs2d)


def _mask_body(adj_ref, inxs_ref, mask_ref, *, blk):
    ix = inxs_ref[...]
    lo = ix & (_D - 1)
    hi = ix >> 7
    acc = jnp.zeros((blk, _K), jnp.float32)
    for p in range(4):
        packed = jnp.zeros((blk, _D), jnp.float32)
        for m in range(16):
            c = p * 16 + m
            sg = adj_ref[:, c * _D:(c + 1) * _D] > 0
            packed = packed + jnp.where(sg, float(1 << m), 0.0)
        g = jnp.take_along_axis(packed, lo, axis=-1)
        acc = jnp.where((hi >> 4) == p, g, acc)
    bits = (acc.astype(jnp.int32) >> (hi & 15)) & 1
    mask_ref[...] = jnp.where(bits == 1, 0.0, -1e22).astype(jnp.float32)


def _mask_extract(adj, inxs, interpret=False):
    blk = 128
    return pl.pallas_call(
        functools.partial(_mask_body, blk=blk),
        grid=(_N // blk,),
        in_specs=[
            pl.BlockSpec((blk, _N), lambda i: (i, 0)),
            pl.BlockSpec((blk, _K), lambda i: (i, 0)),
        ],
        out_specs=pl.BlockSpec((blk, _K), lambda i: (i, 0)),
        out_shape=jax.ShapeDtypeStruct((_N, _K), jnp.float32),
        interpret=interpret,
    )(adj, inxs)


_INV_SQRT_D = 1.0 / math.sqrt(_D)


def _attn_body(x_ref, kvn_ref, mask_ref, rep_ref, rept_ref, oh_ref,
               wq_ref, wo_ref, l1g_ref, l1b_ref, w1_ref, b1_ref,
               w2_ref, b2_ref, l2g_ref, l2b_ref, out_ref, *, blk):
    f32 = jnp.float32
    dot = functools.partial(jnp.dot, preferred_element_type=f32)
    xb = x_ref[...]
    q = dot(xb, wq_ref[...])
    wkv = kvn_ref[...]
    kn = lax.bitcast_convert_type(wkv << 16, f32)
    vn = lax.bitcast_convert_type(wkv & jnp.int32(-65536), f32)
    bf16 = jnp.bfloat16
    rep = rep_ref[...]
    rept = rept_ref[...]
    oh = oh_ref[...]
    ones_d_k = jnp.ones((_D, _K), bf16)
    ones_k_d = jnp.ones((_K, _D), bf16)
    q_rep = dot(rep, q.astype(bf16))
    e = q_rep * kn
    e1 = dot(e.astype(bf16), ones_d_k)
    scores = dot(rept, (e1 * oh).astype(bf16)) * _INV_SQRT_D
    s = scores + mask_ref[...]
    m = jnp.max(s, axis=-1, keepdims=True)
    ex = jnp.exp(s - m)
    attn = ex / jnp.sum(ex, axis=-1, keepdims=True)
    a1 = dot(rep, attn.astype(bf16))
    attn_rep = dot((a1 * oh).astype(bf16), ones_k_d)
    w = attn_rep * vn
    att = jnp.sum(w.reshape(blk, _K, _D), axis=1)
    att = dot(att, wo_ref[...])
    h = xb + jnp.maximum(att, 0.0)
    mu = jnp.mean(h, axis=-1, keepdims=True)
    var = jnp.mean((h - mu) ** 2, axis=-1, keepdims=True)
    h = (h - mu) / jnp.sqrt(var + 1e-5) * l1g_ref[...] + l1b_ref[...]
    f = jnp.maximum(dot(h, w1_ref[...]) + b1_ref[...], 0.0)
    f = dot(f, w2_ref[...]) + b2_ref[...]
    h2 = h + f
    mu2 = jnp.mean(h2, axis=-1, keepdims=True)
    var2 = jnp.mean((h2 - mu2) ** 2, axis=-1, keepdims=True)
    out_ref[...] = ((h2 - mu2) / jnp.sqrt(var2 + 1e-5) * l2g_ref[...]
                    + l2b_ref[...])


def _attn_ffn(x, kvn_flat, mask, Wq, Wo, ln1_g, ln1_b,
              W1, b1, W2, b2, ln2_g, ln2_b, interpret=False):
    blk = 128
    r = jnp.arange(blk * _K, dtype=jnp.int32)
    rep = (r[:, None] // _K == jnp.arange(blk)[None, :]).astype(jnp.bfloat16)
    rept = rep.T
    oh = (r[:, None] % _K == jnp.arange(_K)[None, :]).astype(jnp.bfloat16)
    const = lambda i: (0, 0)
    return pl.pallas_call(
        functools.partial(_attn_body, blk=blk),
        grid=(_N // blk,),
        in_specs=[
            pl.BlockSpec((blk, _D), lambda i: (i, 0)),
            pl.BlockSpec((blk * _K, _D), lambda i: (i, 0)),
            pl.BlockSpec((blk, _K), lambda i: (i, 0)),
            pl.BlockSpec((blk * _K, blk), const),
            pl.BlockSpec((blk, blk * _K), const),
            pl.BlockSpec((blk * _K, _K), const),
            pl.BlockSpec((_D, _D), const),
            pl.BlockSpec((_D, _D), const),
            pl.BlockSpec((1, _D), const),
            pl.BlockSpec((1, _D), const),
            pl.BlockSpec((_D, _DFF), const),
            pl.BlockSpec((1, _DFF), const),
            pl.BlockSpec((_DFF, _D), const),
            pl.BlockSpec((1, _D), const),
            pl.BlockSpec((1, _D), const),
            pl.BlockSpec((1, _D), const),
        ],
        out_specs=pl.BlockSpec((blk, _D), lambda i: (i, 0)),
        out_shape=jax.ShapeDtypeStruct((_N, _D), jnp.float32),
        interpret=interpret,
    )(x, kvn_flat, mask, rep, rept, oh, Wq, Wo, ln1_g, ln1_b,
      W1, b1, W2, b2, ln2_g, ln2_b)


def kernel(x, adj, inxs, Wq, Wk, Wv, Wo, ln1_g, ln1_b, W1, b1, W2, b2,
           ln2_g, ln2_b):
    adj = jnp.squeeze(adj)
    inxs = inxs.astype(jnp.int32)
    kv = _project_kv(x, Wk, Wv)
    inxs2d = inxs.reshape(_N * _K // _IDXW, _IDXW)
    mask = _mask_extract(adj, inxs)
    kvn_flat = _sc_gather(kv, inxs2d)
    return _attn_ffn(x, kvn_flat, mask,
                     Wq, Wo, ln1_g.reshape(1, _D), ln1_b.reshape(1, _D),
                     W1, b1.reshape(1, _DFF), W2, b2.reshape(1, _D),
                     ln2_g.reshape(1, _D), ln2_b.reshape(1, _D))


# speedup vs baseline: 5.7457x; 1.1967x over previous
import functools
import math

import jax
import jax.numpy as jnp
from jax import lax
from jax.experimental import pallas as pl
from jax.experimental.pallas import tpu as pltpu
from jax.experimental.pallas import tpu_sc as plsc

_N = 8192
_D = 128
_K = 32
_DFF = int(_D * 1.5)

_NC = 2
_NS = 16
_NW = _NC * _NS
_NODES_W = _N // _NW
_CH_NODES = 8
_CH_PAIRS = _CH_NODES * _K
_N_CH = _NODES_W // _CH_NODES
_IDXW = 128


def _kv_body(x_ref, wk_ref, wv_ref, kv_ref):
    xb = x_ref[...]
    k = jnp.dot(xb, wk_ref[...], preferred_element_type=jnp.float32)
    v = jnp.dot(xb, wv_ref[...], preferred_element_type=jnp.float32)
    ki = lax.bitcast_convert_type(k.astype(jnp.bfloat16),
                                  jnp.int16).astype(jnp.int32)
    vi = lax.bitcast_convert_type(v.astype(jnp.bfloat16),
                                  jnp.int16).astype(jnp.int32)
    kv_ref[...] = (ki & 0xFFFF) | (vi << 16)


def _project_kv(x, Wk, Wv):
    bp = 1024
    return pl.pallas_call(
        _kv_body,
        grid=(_N // bp,),
        in_specs=[
            pl.BlockSpec((bp, _D), lambda i: (i, 0)),
            pl.BlockSpec((_D, _D), lambda i: (0, 0)),
            pl.BlockSpec((_D, _D), lambda i: (0, 0)),
        ],
        out_specs=pl.BlockSpec((bp, _D), lambda i: (i, 0)),
        out_shape=jax.ShapeDtypeStruct((_N, _D), jnp.int32),
    )(x, Wk, Wv)


_ROWS_W = _NODES_W * _K // _IDXW
_CH_ROWS = _CH_PAIRS // _IDXW


def _sc_gather_kernel(kv_hbm, inxs_hbm, kvn_out,
                      idx_all, b0, b1, b2,
                      gsem0, gsem1, gsem2, wsem0, wsem1, wsem2):
    wid = lax.axis_index("s") * _NC + lax.axis_index("c")
    node0 = wid * _NODES_W
    pltpu.sync_copy(inxs_hbm.at[pl.ds(wid * _ROWS_W, _ROWS_W)], idx_all)

    bufs = (b0, b1, b2)
    gsems = (gsem0, gsem1, gsem2)
    wsems = (wsem0, wsem1, wsem2)

    def gmk(g, slot):
        return [pltpu.make_async_copy(
            kv_hbm.at[idx_all.at[g * _CH_ROWS + j]],
            bufs[slot].at[pl.ds(j * _IDXW, _IDXW)],
            gsems[slot]) for j in range(_CH_ROWS)]

    def wmk(g, slot):
        pair0 = (node0 + g * _CH_NODES) * _K
        return pltpu.make_async_copy(
            bufs[slot], kvn_out.at[pl.ds(pair0, _CH_PAIRS)], wsems[slot])

    for cp in gmk(0, 0):
        cp.start()
    for cp in gmk(1, 1):
        cp.start()

    def body(g, carry):
        s = lax.rem(g, 3)

        def run(sl):
            for cp in gmk(g, sl):
                cp.wait()
            wmk(g, sl).start()

            @pl.when(g >= 1)
            def _():
                wmk(g - 1, (sl + 2) % 3).wait()

            @pl.when(g + 2 < _N_CH)
            def _():
                for cp in gmk(g + 2, (sl + 2) % 3):
                    cp.start()

        for sl in range(3):
            @pl.when(s == sl)
            def _():
                run(sl)
        return carry

    lax.fori_loop(0, _N_CH, body, 0)
    wmk(_N_CH - 1, (_N_CH - 1) % 3).wait()


def _sc_gather(kv, inxs2d):
    mesh = plsc.VectorSubcoreMesh(core_axis_name="c", subcore_axis_name="s",
                                  num_cores=_NC, num_subcores=_NS)
    fn = functools.partial(
        pl.kernel,
        out_type=jax.ShapeDtypeStruct((_N * _K, _D), jnp.int32),
        mesh=mesh,
        scratch_types=(
            pltpu.VMEM((_ROWS_W, _IDXW), jnp.int32),
            pltpu.VMEM((_CH_PAIRS, _D), jnp.int32),
            pltpu.VMEM((_CH_PAIRS, _D), jnp.int32),
            pltpu.VMEM((_CH_PAIRS, _D), jnp.int32),
            pltpu.SemaphoreType.DMA,
            pltpu.SemaphoreType.DMA,
            pltpu.SemaphoreType.DMA,
            pltpu.SemaphoreType.DMA,
            pltpu.SemaphoreType.DMA,
            pltpu.SemaphoreType.DMA,
        ),
    )(_sc_gather_kernel)
    return fn(kv, inxs2d)


def _mask_body(adj_ref, inxs_ref, mask_ref, *, blk):
    ix = inxs_ref[...]
    lo = ix & (_D - 1)
    hi = ix >> 7
    acc = jnp.zeros((blk, _K), jnp.float32)
    for p in range(4):
        packed = jnp.zeros((blk, _D), jnp.float32)
        for m in range(16):
            c = p * 16 + m
            sg = adj_ref[:, c * _D:(c + 1) * _D] > 0
            packed = packed + jnp.where(sg, float(1 << m), 0.0)
        g = jnp.take_along_axis(packed, lo, axis=-1)
        acc = jnp.where((hi >> 4) == p, g, acc)
    bits = (acc.astype(jnp.int32) >> (hi & 15)) & 1
    mask_ref[...] = jnp.where(bits == 1, 0.0, -1e22).astype(jnp.float32)


def _mask_extract(adj, inxs, interpret=False):
    blk = 128
    return pl.pallas_call(
        functools.partial(_mask_body, blk=blk),
        grid=(_N // blk,),
        in_specs=[
            pl.BlockSpec((blk, _N), lambda i: (i, 0)),
            pl.BlockSpec((blk, _K), lambda i: (i, 0)),
        ],
        out_specs=pl.BlockSpec((blk, _K), lambda i: (i, 0)),
        out_shape=jax.ShapeDtypeStruct((_N, _K), jnp.float32),
        interpret=interpret,
    )(adj, inxs)


_INV_SQRT_D = 1.0 / math.sqrt(_D)


def _attn_body(x_ref, kvn_ref, adj_ref, ix_ref, rep_ref, rept_ref, oh_ref,
               wq_ref, wo_ref, l1g_ref, l1b_ref, w1_ref, b1_ref,
               w2_ref, b2_ref, l2g_ref, l2b_ref, out_ref, *, blk):
    f32 = jnp.float32
    dot = functools.partial(jnp.dot, preferred_element_type=f32)
    xb = x_ref[...]
    q = dot(xb, wq_ref[...])
    wkv = kvn_ref[...]
    kn = lax.bitcast_convert_type(wkv << 16, f32)
    vn = lax.bitcast_convert_type(wkv & jnp.int32(-65536), f32)
    bf16 = jnp.bfloat16
    rep = rep_ref[...]
    rept = rept_ref[...]
    oh = oh_ref[...]
    ones_d_k = jnp.ones((_D, _K), bf16)
    ones_k_d = jnp.ones((_K, _D), bf16)
    q_rep = dot(rep, q.astype(bf16))
    e = q_rep * kn
    e1 = dot(e.astype(bf16), ones_d_k)
    scores = dot(rept, (e1 * oh).astype(bf16)) * _INV_SQRT_D
    ix = ix_ref[...]
    lo = ix & (_D - 1)
    hi = ix >> 7
    macc = jnp.zeros((blk, _K), f32)
    for p in range(4):
        packed = jnp.zeros((blk, _D), f32)
        for mm in range(16):
            c = p * 16 + mm
            sg = adj_ref[:, c * _D:(c + 1) * _D] > 0
            packed = packed + jnp.where(sg, float(1 << mm), 0.0)
        gth = jnp.take_along_axis(packed, lo, axis=-1)
        macc = jnp.where((hi >> 4) == p, gth, macc)
    bits = (macc.astype(jnp.int32) >> (hi & 15)) & 1
    mask = jnp.where(bits == 1, 0.0, -1e22).astype(f32)
    s = scores + mask
    m = jnp.max(s, axis=-1, keepdims=True)
    ex = jnp.exp(s - m)
    attn = ex / jnp.sum(ex, axis=-1, keepdims=True)
    a1 = dot(rep, attn.astype(bf16))
    attn_rep = dot((a1 * oh).astype(bf16), ones_k_d)
    w = attn_rep * vn
    att = jnp.sum(w.reshape(blk, _K, _D), axis=1)
    att = dot(att, wo_ref[...])
    h = xb + jnp.maximum(att, 0.0)
    mu = jnp.mean(h, axis=-1, keepdims=True)
    var = jnp.mean((h - mu) ** 2, axis=-1, keepdims=True)
    h = (h - mu) / jnp.sqrt(var + 1e-5) * l1g_ref[...] + l1b_ref[...]
    f = jnp.maximum(dot(h, w1_ref[...]) + b1_ref[...], 0.0)
    f = dot(f, w2_ref[...]) + b2_ref[...]
    h2 = h + f
    mu2 = jnp.mean(h2, axis=-1, keepdims=True)
    var2 = jnp.mean((h2 - mu2) ** 2, axis=-1, keepdims=True)
    out_ref[...] = ((h2 - mu2) / jnp.sqrt(var2 + 1e-5) * l2g_ref[...]
                    + l2b_ref[...])


def _attn_ffn(x, kvn_flat, adj, inxs, Wq, Wo, ln1_g, ln1_b,
              W1, b1, W2, b2, ln2_g, ln2_b, interpret=False):
    blk = 128
    r = jnp.arange(blk * _K, dtype=jnp.int32)
    rep = (r[:, None] // _K == jnp.arange(blk)[None, :]).astype(jnp.bfloat16)
    rept = rep.T
    oh = (r[:, None] % _K == jnp.arange(_K)[None, :]).astype(jnp.bfloat16)
    const = lambda i: (0, 0)
    return pl.pallas_call(
        functools.partial(_attn_body, blk=blk),
        grid=(_N // blk,),
        in_specs=[
            pl.BlockSpec((blk, _D), lambda i: (i, 0)),
            pl.BlockSpec((blk * _K, _D), lambda i: (i, 0)),
            pl.BlockSpec((blk, _N), lambda i: (i, 0)),
            pl.BlockSpec((blk, _K), lambda i: (i, 0)),
            pl.BlockSpec((blk * _K, blk), const),
            pl.BlockSpec((blk, blk * _K), const),
            pl.BlockSpec((blk * _K, _K), const),
            pl.BlockSpec((_D, _D), const),
            pl.BlockSpec((_D, _D), const),
            pl.BlockSpec((1, _D), const),
            pl.BlockSpec((1, _D), const),
            pl.BlockSpec((_D, _DFF), const),
            pl.BlockSpec((1, _DFF), const),
            pl.BlockSpec((_DFF, _D), const),
            pl.BlockSpec((1, _D), const),
            pl.BlockSpec((1, _D), const),
            pl.BlockSpec((1, _D), const),
        ],
        out_specs=pl.BlockSpec((blk, _D), lambda i: (i, 0)),
        out_shape=jax.ShapeDtypeStruct((_N, _D), jnp.float32),
        interpret=interpret,
    )(x, kvn_flat, adj, inxs, rep, rept, oh, Wq, Wo, ln1_g, ln1_b,
      W1, b1, W2, b2, ln2_g, ln2_b)


def kernel(x, adj, inxs, Wq, Wk, Wv, Wo, ln1_g, ln1_b, W1, b1, W2, b2,
           ln2_g, ln2_b):
    adj = jnp.squeeze(adj)
    inxs = inxs.astype(jnp.int32)
    kv = _project_kv(x, Wk, Wv)
    inxs2d = inxs.reshape(_N * _K // _IDXW, _IDXW)
    kvn_flat = _sc_gather(kv, inxs2d)
    return _attn_ffn(x, kvn_flat, adj, inxs,
                     Wq, Wo, ln1_g.reshape(1, _D), ln1_b.reshape(1, _D),
                     W1, b1.reshape(1, _DFF), W2, b2.reshape(1, _D),
                     ln2_g.reshape(1, _D), ln2_b.reshape(1, _D))


# speedup vs baseline: 5.9150x; 1.0295x over previous
import functools
import math

import jax
import jax.numpy as jnp
from jax import lax
from jax.experimental import pallas as pl
from jax.experimental.pallas import tpu as pltpu
from jax.experimental.pallas import tpu_sc as plsc

_N = 8192
_D = 128
_K = 32
_DFF = int(_D * 1.5)

_NC = 2
_NS = 16
_NW = _NC * _NS
_NODES_W = _N // _NW
_CH_NODES = 8
_CH_PAIRS = _CH_NODES * _K
_N_CH = _NODES_W // _CH_NODES
_IDXW = 128


def _kv_body(x_ref, wk_ref, wv_ref, kv_ref):
    xb = x_ref[...]
    k = jnp.dot(xb, wk_ref[...], preferred_element_type=jnp.float32)
    v = jnp.dot(xb, wv_ref[...], preferred_element_type=jnp.float32)
    ki = lax.bitcast_convert_type(k.astype(jnp.bfloat16),
                                  jnp.int16).astype(jnp.int32)
    vi = lax.bitcast_convert_type(v.astype(jnp.bfloat16),
                                  jnp.int16).astype(jnp.int32)
    kv_ref[...] = (ki & 0xFFFF) | (vi << 16)


def _project_kv(x, Wk, Wv):
    bp = 1024
    return pl.pallas_call(
        _kv_body,
        grid=(_N // bp,),
        in_specs=[
            pl.BlockSpec((bp, _D), lambda i: (i, 0)),
            pl.BlockSpec((_D, _D), lambda i: (0, 0)),
            pl.BlockSpec((_D, _D), lambda i: (0, 0)),
        ],
        out_specs=pl.BlockSpec((bp, _D), lambda i: (i, 0)),
        out_shape=jax.ShapeDtypeStruct((_N, _D), jnp.int32),
    )(x, Wk, Wv)


_ROWS_W = _NODES_W * _K // _IDXW
_CH_ROWS = _CH_PAIRS // _IDXW


def _sc_gather_kernel(kv_hbm, inxs_hbm, kvn_out,
                      idx_all, b0, b1, b2,
                      gsem0, gsem1, gsem2, wsem0, wsem1, wsem2):
    wid = lax.axis_index("s") * _NC + lax.axis_index("c")
    node0 = wid * _NODES_W
    pltpu.sync_copy(inxs_hbm.at[pl.ds(wid * _ROWS_W, _ROWS_W)], idx_all)

    bufs = (b0, b1, b2)
    gsems = (gsem0, gsem1, gsem2)
    wsems = (wsem0, wsem1, wsem2)

    def gmk(g, slot):
        return [pltpu.make_async_copy(
            kv_hbm.at[idx_all.at[g * _CH_ROWS + j]],
            bufs[slot].at[pl.ds(j * _IDXW, _IDXW)],
            gsems[slot]) for j in range(_CH_ROWS)]

    def wmk(g, slot):
        pair0 = (node0 + g * _CH_NODES) * _K
        return pltpu.make_async_copy(
            bufs[slot], kvn_out.at[pl.ds(pair0, _CH_PAIRS)], wsems[slot])

    for cp in gmk(0, 0):
        cp.start()
    for cp in gmk(1, 1):
        cp.start()

    def body(g, carry):
        s = lax.rem(g, 3)

        def run(sl):
            for cp in gmk(g, sl):
                cp.wait()
            wmk(g, sl).start()

            @pl.when(g >= 1)
            def _():
                wmk(g - 1, (sl + 2) % 3).wait()

            @pl.when(g + 2 < _N_CH)
            def _():
                for cp in gmk(g + 2, (sl + 2) % 3):
                    cp.start()

        for sl in range(3):
            @pl.when(s == sl)
            def _():
                run(sl)
        return carry

    lax.fori_loop(0, _N_CH, body, 0)
    wmk(_N_CH - 1, (_N_CH - 1) % 3).wait()


def _sc_gather(kv, inxs2d):
    mesh = plsc.VectorSubcoreMesh(core_axis_name="c", subcore_axis_name="s",
                                  num_cores=_NC, num_subcores=_NS)
    fn = functools.partial(
        pl.kernel,
        out_type=jax.ShapeDtypeStruct((_N * _K, _D), jnp.int32),
        mesh=mesh,
        scratch_types=(
            pltpu.VMEM((_ROWS_W, _IDXW), jnp.int32),
            pltpu.VMEM((_CH_PAIRS, _D), jnp.int32),
            pltpu.VMEM((_CH_PAIRS, _D), jnp.int32),
            pltpu.VMEM((_CH_PAIRS, _D), jnp.int32),
            pltpu.SemaphoreType.DMA,
            pltpu.SemaphoreType.DMA,
            pltpu.SemaphoreType.DMA,
            pltpu.SemaphoreType.DMA,
            pltpu.SemaphoreType.DMA,
            pltpu.SemaphoreType.DMA,
        ),
    )(_sc_gather_kernel)
    return fn(kv, inxs2d)


def _mask_body(adj_ref, inxs_ref, mask_ref, *, blk):
    ix = inxs_ref[...]
    lo = ix & (_D - 1)
    hi = ix >> 7
    acc = jnp.zeros((blk, _K), jnp.float32)
    for p in range(4):
        packed = jnp.zeros((blk, _D), jnp.float32)
        for m in range(16):
            c = p * 16 + m
            sg = adj_ref[:, c * _D:(c + 1) * _D] > 0
            packed = packed + jnp.where(sg, float(1 << m), 0.0)
        g = jnp.take_along_axis(packed, lo, axis=-1)
        acc = jnp.where((hi >> 4) == p, g, acc)
    bits = (acc.astype(jnp.int32) >> (hi & 15)) & 1
    mask_ref[...] = jnp.where(bits == 1, 0.0, -1e22).astype(jnp.float32)


def _mask_extract(adj, inxs, interpret=False):
    blk = 128
    return pl.pallas_call(
        functools.partial(_mask_body, blk=blk),
        grid=(_N // blk,),
        in_specs=[
            pl.BlockSpec((blk, _N), lambda i: (i, 0)),
            pl.BlockSpec((blk, _K), lambda i: (i, 0)),
        ],
        out_specs=pl.BlockSpec((blk, _K), lambda i: (i, 0)),
        out_shape=jax.ShapeDtypeStruct((_N, _K), jnp.float32),
        interpret=interpret,
    )(adj, inxs)


_INV_SQRT_D = 1.0 / math.sqrt(_D)


def _attn_body(x_ref, kvn_ref, adj_ref, ix_ref, rep_ref, oh_ref,
               wq_ref, wo_ref, l1g_ref, l1b_ref, w1_ref, b1_ref,
               w2_ref, b2_ref, l2g_ref, l2b_ref, out_ref, *, blk):
    f32 = jnp.float32
    dot = functools.partial(jnp.dot, preferred_element_type=f32)
    xb = x_ref[...]
    q = dot(xb, wq_ref[...])
    wkv = kvn_ref[...]
    kn = lax.bitcast_convert_type(wkv << 16, f32)
    vn = lax.bitcast_convert_type(wkv & jnp.int32(-65536), f32)
    bf16 = jnp.bfloat16
    rep = rep_ref[...]
    oh = oh_ref[...]
    ones_d_k = jnp.ones((_D, _K), bf16)
    ones_k_d = jnp.ones((_K, _D), bf16)
    q_rep = dot(rep, q.astype(bf16))
    e = q_rep * kn
    e1 = dot(e.astype(bf16), ones_d_k)
    scores = jnp.sum((e1 * oh).reshape(blk, _K, _K),
                     axis=1) * _INV_SQRT_D
    ix = ix_ref[...]
    lo = ix & (_D - 1)
    hi = ix >> 7
    macc = jnp.zeros((blk, _K), f32)
    for p in range(4):
        packed = jnp.zeros((blk, _D), f32)
        for mm in range(16):
            c = p * 16 + mm
            sg = adj_ref[:, c * _D:(c + 1) * _D] > 0
            packed = packed + jnp.where(sg, float(1 << mm), 0.0)
        gth = jnp.take_along_axis(packed, lo, axis=-1)
        macc = jnp.where((hi >> 4) == p, gth, macc)
    bits = (macc.astype(jnp.int32) >> (hi & 15)) & 1
    mask = jnp.where(bits == 1, 0.0, -1e22).astype(f32)
    s = scores + mask
    m = jnp.max(s, axis=-1, keepdims=True)
    ex = jnp.exp(s - m)
    attn = ex / jnp.sum(ex, axis=-1, keepdims=True)
    a1 = dot(rep, attn.astype(bf16))
    attn_rep = dot((a1 * oh).astype(bf16), ones_k_d)
    w = attn_rep * vn
    att = jnp.sum(w.reshape(blk, _K, _D), axis=1)
    att = dot(att, wo_ref[...])
    h = xb + jnp.maximum(att, 0.0)
    mu = jnp.mean(h, axis=-1, keepdims=True)
    var = jnp.mean((h - mu) ** 2, axis=-1, keepdims=True)
    h = (h - mu) / jnp.sqrt(var + 1e-5) * l1g_ref[...] + l1b_ref[...]
    f = jnp.maximum(dot(h, w1_ref[...]) + b1_ref[...], 0.0)
    f = dot(f, w2_ref[...]) + b2_ref[...]
    h2 = h + f
    mu2 = jnp.mean(h2, axis=-1, keepdims=True)
    var2 = jnp.mean((h2 - mu2) ** 2, axis=-1, keepdims=True)
    out_ref[...] = ((h2 - mu2) / jnp.sqrt(var2 + 1e-5) * l2g_ref[...]
                    + l2b_ref[...])


def _attn_ffn(x, kvn_flat, adj, inxs, Wq, Wo, ln1_g, ln1_b,
              W1, b1, W2, b2, ln2_g, ln2_b, interpret=False):
    blk = 128
    r = jnp.arange(blk * _K, dtype=jnp.int32)
    rep = (r[:, None] // _K == jnp.arange(blk)[None, :]).astype(jnp.bfloat16)
    oh = (r[:, None] % _K == jnp.arange(_K)[None, :]).astype(jnp.bfloat16)
    const = lambda i: (0, 0)
    return pl.pallas_call(
        functools.partial(_attn_body, blk=blk),
        grid=(_N // blk,),
        in_specs=[
            pl.BlockSpec((blk, _D), lambda i: (i, 0)),
            pl.BlockSpec((blk * _K, _D), lambda i: (i, 0)),
            pl.BlockSpec((blk, _N), lambda i: (i, 0)),
            pl.BlockSpec((blk, _K), lambda i: (i, 0)),
            pl.BlockSpec((blk * _K, blk), const),
            pl.BlockSpec((blk * _K, _K), const),
            pl.BlockSpec((_D, _D), const),
            pl.BlockSpec((_D, _D), const),
            pl.BlockSpec((1, _D), const),
            pl.BlockSpec((1, _D), const),
            pl.BlockSpec((_D, _DFF), const),
            pl.BlockSpec((1, _DFF), const),
            pl.BlockSpec((_DFF, _D), const),
            pl.BlockSpec((1, _D), const),
            pl.BlockSpec((1, _D), const),
            pl.BlockSpec((1, _D), const),
        ],
        out_specs=pl.BlockSpec((blk, _D), lambda i: (i, 0)),
        out_shape=jax.ShapeDtypeStruct((_N, _D), jnp.float32),
        interpret=interpret,
    )(x, kvn_flat, adj, inxs, rep, oh, Wq, Wo, ln1_g, ln1_b,
      W1, b1, W2, b2, ln2_g, ln2_b)


def kernel(x, adj, inxs, Wq, Wk, Wv, Wo, ln1_g, ln1_b, W1, b1, W2, b2,
           ln2_g, ln2_b):
    adj = jnp.squeeze(adj)
    inxs = inxs.astype(jnp.int32)
    kv = _project_kv(x, Wk, Wv)
    inxs2d = inxs.reshape(_N * _K // _IDXW, _IDXW)
    kvn_flat = _sc_gather(kv, inxs2d)
    return _attn_ffn(x, kvn_flat, adj, inxs,
                     Wq, Wo, ln1_g.reshape(1, _D), ln1_b.reshape(1, _D),
                     W1, b1.reshape(1, _DFF), W2, b2.reshape(1, _D),
                     ln2_g.reshape(1, _D), ln2_b.reshape(1, _D))
